# baseline (speedup 1.0000x reference)
import jax
import jax.numpy as jnp
from jax.experimental import pallas as pl
from jax.experimental.pallas import tpu as pltpu

_TM = 512
_TC = 512
_KC = 256


def _fused_mlp_kernel(x_ref, w1_hbm, b1_ref, w2_hbm, b2_ref, s_ref, o_ref,
                      w1b_ref, w2b_ref, st1_ref, st2_ref, sem1, sem2):
    m = pl.program_id(0)
    V = w1b_ref.shape[1]
    n_c = V // _TC

    @pl.when(m == 0)
    def _():
        K = w1b_ref.shape[0]
        n_k = K // _KC

        def cp1(r):
            return pltpu.make_async_copy(
                w1_hbm.at[pl.ds(r * _KC, _KC), :],
                st1_ref.at[r % 2], sem1.at[r % 2])

        def cp2(c):
            return pltpu.make_async_copy(
                w2_hbm.at[pl.ds(c * _TC, _TC), :],
                st2_ref.at[c % 2], sem2.at[c % 2])

        cp1(0).start()
        cp1(1).start()
        x = x_ref[...]
        xb = x.astype(jnp.bfloat16)
        hacc = jnp.broadcast_to(b1_ref[...], (x.shape[0], V))
        for r in range(n_k):
            ks = pl.ds(r * _KC, _KC)
            cp1(r).wait()
            w1b_ref[ks, :] = st1_ref[r % 2].astype(jnp.bfloat16)
            if r + 2 < n_k:
                cp1(r + 2).start()
            elif r + 2 - n_k < 2:
                cp2(r + 2 - n_k).start()
            hacc = hacc + jnp.dot(xb[:, r * _KC:(r + 1) * _KC],
                                  w1b_ref[ks, :],
                                  preferred_element_type=jnp.float32)
        h = jnp.maximum(hacc, 0.0).astype(jnp.bfloat16)
        acc = x + b2_ref[...]
        for c in range(n_c):
            sl = pl.ds(c * _TC, _TC)
            cp2(c).wait()
            w2b_ref[sl, :] = st2_ref[c % 2].astype(jnp.bfloat16)
            if c + 2 < n_c:
                cp2(c + 2).start()
            acc = acc + jnp.dot(h[:, c * _TC:(c + 1) * _TC],
                                w2b_ref[sl, :],
                                preferred_element_type=jnp.float32)
        var = jnp.mean(acc * acc, axis=-1, keepdims=True)
        o_ref[...] = (acc * jax.lax.rsqrt(var + 1e-6)) * s_ref[...]

    @pl.when(m > 0)
    def _():
        x = x_ref[...]
        h = jnp.dot(x.astype(jnp.bfloat16), w1b_ref[...],
                    preferred_element_type=jnp.float32)
        h = jnp.maximum(h + b1_ref[...], 0.0).astype(jnp.bfloat16)
        y = jnp.dot(h, w2b_ref[...], preferred_element_type=jnp.float32)
        y = y + b2_ref[...] + x
        var = jnp.mean(y * y, axis=-1, keepdims=True)
        o_ref[...] = (y * jax.lax.rsqrt(var + 1e-6)) * s_ref[...]


def kernel(x, W1, b1, W2, b2, ln_scale):
    B, S, D = x.shape
    K, V = W1.shape
    M = B * S
    n_m = M // _TM

    xf = x.reshape(M, D)
    b1r = b1.reshape(1, V)
    b2r = b2.reshape(1, D)
    snr = ln_scale.reshape(1, D)

    out = pl.pallas_call(
        _fused_mlp_kernel,
        grid=(n_m,),
        in_specs=[
            pl.BlockSpec((_TM, D), lambda m: (m, 0)),
            pl.BlockSpec(memory_space=pltpu.MemorySpace.HBM),
            pl.BlockSpec((1, V), lambda m: (0, 0)),
            pl.BlockSpec(memory_space=pltpu.MemorySpace.HBM),
            pl.BlockSpec((1, D), lambda m: (0, 0)),
            pl.BlockSpec((1, D), lambda m: (0, 0)),
        ],
        out_specs=pl.BlockSpec((_TM, D), lambda m: (m, 0)),
        out_shape=jax.ShapeDtypeStruct((M, D), jnp.float32),
        scratch_shapes=[
            pltpu.VMEM((K, V), jnp.bfloat16),
            pltpu.VMEM((V, D), jnp.bfloat16),
            pltpu.VMEM((2, _KC, V), jnp.float32),
            pltpu.VMEM((2, _TC, D), jnp.float32),
            pltpu.SemaphoreType.DMA((2,)),
            pltpu.SemaphoreType.DMA((2,)),
        ],
        compiler_params=pltpu.CompilerParams(
            dimension_semantics=("arbitrary",),
        ),
    )(xf, W1, b1r, W2, b2r, snr)
    return out.reshape(B, S, D)


# speedup vs baseline: 1.0281x; 1.0281x over previous
import jax
import jax.numpy as jnp
from jax.experimental import pallas as pl
from jax.experimental.pallas import tpu as pltpu

_TM = 512
_TC = 512
_TC1 = 1024


def _fused_mlp_kernel(x_ref, w1_hbm, b1_ref, w2_hbm, b2_ref, s_ref, o_ref,
                      w1b_ref, w2b_ref, st1_ref, st2_ref, sem1, sem2):
    m = pl.program_id(0)
    V = w1b_ref.shape[1]
    n_c = V // _TC

    @pl.when(m == 0)
    def _():
        n_c1 = V // _TC1
        r = _TC1 // _TC

        def cp1(c):
            return pltpu.make_async_copy(
                w1_hbm.at[:, pl.ds(c * _TC1, _TC1)],
                st1_ref.at[c % 2], sem1.at[c % 2])

        def cp2(c):
            return pltpu.make_async_copy(
                w2_hbm.at[pl.ds(c * _TC, _TC), :],
                st2_ref.at[c % 2], sem2.at[c % 2])

        cp1(0).start()
        cp2(0).start()
        cp1(1).start()
        cp2(1).start()
        x = x_ref[...]
        xb = x.astype(jnp.bfloat16)
        acc = x + b2_ref[...]
        for c1 in range(n_c1):
            sl1 = pl.ds(c1 * _TC1, _TC1)
            cp1(c1).wait()
            w1b_ref[:, sl1] = st1_ref[c1 % 2].astype(jnp.bfloat16)
            if c1 + 2 < n_c1:
                cp1(c1 + 2).start()
            h = jnp.dot(xb, w1b_ref[:, sl1],
                        preferred_element_type=jnp.float32)
            h = jnp.maximum(h + b1_ref[:, sl1], 0.0).astype(jnp.bfloat16)
            for j in range(r):
                c = c1 * r + j
                sl = pl.ds(c * _TC, _TC)
                cp2(c).wait()
                w2b_ref[sl, :] = st2_ref[c % 2].astype(jnp.bfloat16)
                if c + 2 < n_c:
                    cp2(c + 2).start()
                acc = acc + jnp.dot(h[:, j * _TC:(j + 1) * _TC],
                                    w2b_ref[sl, :],
                                    preferred_element_type=jnp.float32)
        var = jnp.mean(acc * acc, axis=-1, keepdims=True)
        o_ref[...] = (acc * jax.lax.rsqrt(var + 1e-6)) * s_ref[...]

    @pl.when(m > 0)
    def _():
        x = x_ref[...]
        h = jnp.dot(x.astype(jnp.bfloat16), w1b_ref[...],
                    preferred_element_type=jnp.float32)
        h = jnp.maximum(h + b1_ref[...], 0.0).astype(jnp.bfloat16)
        y = jnp.dot(h, w2b_ref[...], preferred_element_type=jnp.float32)
        y = y + b2_ref[...] + x
        var = jnp.mean(y * y, axis=-1, keepdims=True)
        o_ref[...] = (y * jax.lax.rsqrt(var + 1e-6)) * s_ref[...]


def kernel(x, W1, b1, W2, b2, ln_scale):
    B, S, D = x.shape
    K, V = W1.shape
    M = B * S
    n_m = M // _TM

    xf = x.reshape(M, D)
    b1r = b1.reshape(1, V)
    b2r = b2.reshape(1, D)
    snr = ln_scale.reshape(1, D)

    out = pl.pallas_call(
        _fused_mlp_kernel,
        grid=(n_m,),
        in_specs=[
            pl.BlockSpec((_TM, D), lambda m: (m, 0)),
            pl.BlockSpec(memory_space=pltpu.MemorySpace.HBM),
            pl.BlockSpec((1, V), lambda m: (0, 0)),
            pl.BlockSpec(memory_space=pltpu.MemorySpace.HBM),
            pl.BlockSpec((1, D), lambda m: (0, 0)),
            pl.BlockSpec((1, D), lambda m: (0, 0)),
        ],
        out_specs=pl.BlockSpec((_TM, D), lambda m: (m, 0)),
        out_shape=jax.ShapeDtypeStruct((M, D), jnp.float32),
        scratch_shapes=[
            pltpu.VMEM((K, V), jnp.bfloat16),
            pltpu.VMEM((V, D), jnp.bfloat16),
            pltpu.VMEM((2, K, _TC1), jnp.float32),
            pltpu.VMEM((2, _TC, D), jnp.float32),
            pltpu.SemaphoreType.DMA((2,)),
            pltpu.SemaphoreType.DMA((2,)),
        ],
        compiler_params=pltpu.CompilerParams(
            dimension_semantics=("arbitrary",),
        ),
    )(xf, W1, b1r, W2, b2r, snr)
    return out.reshape(B, S, D)
